# baseline (speedup 1.0000x reference)
import functools

import jax
import jax.numpy as jnp
from jax import lax
from jax.experimental import pallas as pl
from jax.experimental.pallas import tpu as pltpu
from jax.experimental.pallas import tpu_sc as plsc

D = 64
L = 16
W = 128
NBUF = 4


def _recommender_scores(uids, iids, utabT, itabT, utail, itail, *,
                        n_workers, b_per_w):
    mesh = plsc.VectorSubcoreMesh(core_axis_name="c", subcore_axis_name="s")

    @functools.partial(
        pl.kernel,
        mesh=mesh,
        compiler_params=pltpu.CompilerParams(needs_layout_passes=False),
        out_type=jax.ShapeDtypeStruct((n_workers, b_per_w), jnp.float32),
        scratch_types=[
            pltpu.VMEM((b_per_w,), jnp.int32),
            pltpu.VMEM((b_per_w,), jnp.int32),
            pltpu.VMEM((D,), jnp.int32),
            pltpu.VMEM((D, W), jnp.float32),
            pltpu.VMEM((D, W), jnp.float32),
            pltpu.VMEM((NBUF, D, W), jnp.float32),
            pltpu.VMEM((NBUF, D, W), jnp.float32),
            pltpu.VMEM((L * b_per_w,), jnp.float32),
            pltpu.VMEM((b_per_w,), jnp.float32),
            pltpu.SemaphoreType.DMA,
            pltpu.SemaphoreType.DMA,
            pltpu.SemaphoreType.DMA,
            pltpu.SemaphoreType.DMA,
        ],
    )
    def k(uid_hbm, iid_hbm, utab_hbm, itab_hbm, utail_hbm, itail_hbm,
          out_hbm, uids_v, iids_v, fidx_v, utail_v, itail_v, uwin, iwin,
          tpose_v, out_v, sem0, sem1, sem2, sem3):
        wid = lax.axis_index("s") * mesh.num_cores + lax.axis_index("c")
        pltpu.sync_copy(uid_hbm.at[wid], uids_v)
        pltpu.sync_copy(iid_hbm.at[wid], iids_v)
        pltpu.sync_copy(utail_hbm, utail_v)
        pltpu.sync_copy(itail_hbm, itail_v)
        for c in range(D // L):
            fidx_v[pl.ds(c * L, L)] = lax.iota(jnp.int32, L) + c * L
        sems = (sem0, sem1, sem2, sem3)
        lane_ids = lax.iota(jnp.int32, L)

        n_cols = utab_hbm.shape[1]
        tail = (n_cols // W) * W
        last_base = tail - W

        def bases(uvec, ivec, k_):
            ub = pl.multiple_of(
                jnp.minimum(uvec[k_] >> 7, last_base // W) * W, W)
            ib = pl.multiple_of(
                jnp.minimum(ivec[k_] >> 7, last_base // W) * W, W)
            return ub, ib

        def fire(uvec, ivec, k_):
            s = k_ % NBUF
            ub, ib = bases(uvec, ivec, k_)
            pltpu.async_copy(utab_hbm.at[:, pl.ds(ub, W)],
                             uwin.at[s], sems[s])
            pltpu.async_copy(itab_hbm.at[:, pl.ds(ib, W)],
                             iwin.at[s], sems[s])

        def drain(uvec, ivec, k_):
            s = k_ % NBUF
            ub, ib = bases(uvec, ivec, k_)
            pltpu.make_async_copy(utab_hbm.at[:, pl.ds(ub, W)],
                                  uwin.at[s], sems[s]).wait()
            pltpu.make_async_copy(itab_hbm.at[:, pl.ds(ib, W)],
                                  iwin.at[s], sems[s]).wait()

        def pick(win_s, tail_v, idv):
            full_id = jnp.full((L,), idv, jnp.int32)
            lcol = full_id & (W - 1)
            tcol = jnp.maximum(full_id - tail, 0)
            in_tail = full_id >= tail
            def chunk(c):
                wv = plsc.load_gather(win_s, [lane_ids + c * L, lcol])
                tv = plsc.load_gather(tail_v, [lane_ids + c * L, tcol])
                return jnp.where(in_tail, tv, wv)
            return chunk

        def compute(uvec, ivec, p, k_):
            s = k_ % NBUF
            uchunk = pick(uwin.at[s], utail_v, uvec[k_])
            ichunk = pick(iwin.at[s], itail_v, ivec[k_])
            acc = uchunk(0) * ichunk(0)
            for c in range(1, D // L):
                acc += uchunk(c) * ichunk(c)
            plsc.store_scatter(tpose_v, [lane_ids * b_per_w + p], acc)

        n_groups = b_per_w // L
        uvec0 = uids_v[pl.ds(0, L)]
        ivec0 = iids_v[pl.ds(0, L)]
        for k_ in range(NBUF - 1):
            fire(uvec0, ivec0, k_)

        @pl.loop(0, n_groups)
        def body(g):
            uvec = uids_v[pl.ds(g * L, L)]
            ivec = iids_v[pl.ds(g * L, L)]
            for k_ in range(L):
                ahead = k_ + NBUF - 1
                if ahead < L:
                    fire(uvec, ivec, ahead)
                else:
                    @pl.when(g + 1 < n_groups)
                    def _():
                        uv2 = uids_v[pl.ds((g + 1) * L, L)]
                        iv2 = iids_v[pl.ds((g + 1) * L, L)]
                        fire(uv2, iv2, ahead - L)

                drain(uvec, ivec, k_)
                compute(uvec, ivec, g * L + k_, k_)

        @plsc.parallel_loop(0, b_per_w // L, 1, unroll=2)
        def reduce_body(m):
            acc = tpose_v[pl.ds(m * L, L)]
            for c in range(1, L):
                acc += tpose_v[pl.ds(c * b_per_w + m * L, L)]
            out_v[pl.ds(m * L, L)] = acc

        pltpu.sync_copy(out_v, out_hbm.at[wid])

    return k(uids, iids, utabT, itabT, utail, itail)


def _tail_pad(tabT):
    n_cols = tabT.shape[1]
    tail = (n_cols // W) * W
    t = tabT[:, tail:]
    return jnp.pad(t, ((0, 0), (0, W - t.shape[1])))


def kernel(user_ids, item_ids, user_table, item_table):
    b = user_ids.shape[0]
    info = plsc.get_sparse_core_info()
    n_workers = info.num_cores * info.num_subcores
    b_per_w = b // n_workers
    uids = user_ids.astype(jnp.int32).reshape(n_workers, b_per_w)
    iids = item_ids.astype(jnp.int32).reshape(n_workers, b_per_w)
    utabT, itabT = user_table.T, item_table.T
    out = _recommender_scores(uids, iids, utabT, itabT,
                              _tail_pad(utabT), _tail_pad(itabT),
                              n_workers=n_workers, b_per_w=b_per_w)
    return out.reshape(b)
